# baseline (speedup 1.0000x reference)
import functools

import jax
import jax.numpy as jnp
from jax import lax
from jax.experimental import pallas as pl
from jax.experimental.pallas import tpu as pltpu
from jax.experimental.pallas import tpu_sc as plsc

L = 3
EMB = 128
N_NODE = 10000
N_NET = 10000
E = 320000
OUT = 8

NW = 32
CHUNK = 128
CPW = 81
EPW = CPW * CHUNK
NBUF = 3
STRIPE = 624


def _leaky(x):
    return jnp.where(x >= 0, x, 0.01 * x)


def _spmm_body(h_hbm, src_hbm, dst_hbm, w_hbm, zeros_hbm, out_hbm,
               acc, rows0, rows1, rows2, sc0, sc1, sc2, dc0, dc1, dc2,
               wc0, wc1, wc2, gs0, gs1, gs2, ss0, ss1, ss2, ps0, ps1, ps2):
    cid = lax.axis_index("c")
    sid = lax.axis_index("s")
    wid = cid * 16 + sid
    bufs = (rows0, rows1, rows2)
    srcs = (sc0, sc1, sc2)
    dsts = (dc0, dc1, dc2)
    ws = (wc0, wc1, wc2)
    gsems = (gs0, gs1, gs2)
    ssems = (ss0, ss1, ss2)
    psems = (ps0, ps1, ps2)

    @pl.when(sid < 15)
    def _():
        pltpu.sync_copy(zeros_hbm.at[pl.ds(sid * STRIPE, STRIPE)],
                        acc.at[pl.ds(sid * STRIPE, STRIPE)])

    @pl.when(sid == 15)
    def _():
        pltpu.sync_copy(zeros_hbm.at[pl.ds(15 * STRIPE, N_NODE - 15 * STRIPE)],
                        acc.at[pl.ds(15 * STRIPE, N_NODE - 15 * STRIPE)])

    plsc.subcore_barrier()

    def p_start(i, k):
        pltpu.async_copy(src_hbm.at[wid, i], srcs[k], psems[k])
        pltpu.async_copy(dst_hbm.at[wid, i], dsts[k], psems[k])
        pltpu.async_copy(w_hbm.at[wid, i], ws[k], psems[k])

    def p_wait(k):
        pltpu.make_async_copy(src_hbm.at[wid, 0], srcs[k], psems[k]).wait()
        pltpu.make_async_copy(dst_hbm.at[wid, 0], dsts[k], psems[k]).wait()
        pltpu.make_async_copy(w_hbm.at[wid, 0], ws[k], psems[k]).wait()

    def g_start(k):
        pltpu.async_copy(h_hbm.at[srcs[k]], bufs[k], gsems[k])

    def g_wait(k):
        pltpu.make_async_copy(h_hbm.at[srcs[k]], bufs[k], gsems[k]).wait()

    def s_start(k):
        pltpu.async_copy(bufs[k], acc.at[dsts[k]], ssems[k], add=True)

    def s_wait(k):
        pltpu.make_async_copy(bufs[k], acc.at[dsts[k]], ssems[k]).wait()

    def compute(k):
        buf = bufs[k]
        w_c = ws[k]

        def grp_body(g, c2):
            wg = w_c[pl.ds(g * 16, 16)]
            for e16 in range(16):
                e = g * 16 + e16
                wb = wg[e16]
                for cb in range(EMB // 16):
                    sl = pl.ds(cb * 16, 16)
                    buf[e, sl] = buf[e, sl] * wb
            return c2
        lax.fori_loop(0, CHUNK // 16, grp_body, 0)

    p_start(0, 0)
    p_wait(0)
    g_start(0)
    p_start(1, 1)
    n_j = CPW // NBUF

    def jbody(j, carry):
        for k in range(NBUF):
            i = NBUF * j + k
            k1 = (k + 1) % NBUF
            k2 = (k + 2) % NBUF
            if k < 2:
                p_wait(k1)
                g_start(k1)
            else:
                @pl.when(j < n_j - 1)
                def _():
                    p_wait(k1)
                    g_start(k1)
            g_wait(k)
            compute(k)
            s_start(k)
            if k == 0:
                @pl.when(j > 0)
                def _():
                    s_wait(k2)
                p_start(i + 2, k2)
            else:
                @pl.when(j < n_j - 1)
                def _():
                    s_wait(k2)
                    p_start(i + 2, k2)
        return carry

    lax.fori_loop(0, n_j, jbody, 0)
    for k in range(NBUF):
        s_wait(k)
    plsc.subcore_barrier()

    @pl.when(sid < 15)
    def _():
        pltpu.sync_copy(acc.at[pl.ds(sid * STRIPE, STRIPE)],
                        out_hbm.at[cid, pl.ds(sid * STRIPE, STRIPE)])

    @pl.when(sid == 15)
    def _():
        pltpu.sync_copy(acc.at[pl.ds(15 * STRIPE, N_NODE - 15 * STRIPE)],
                        out_hbm.at[cid, pl.ds(15 * STRIPE, N_NODE - 15 * STRIPE)])


_spmm = pl.kernel(
    _spmm_body,
    mesh=plsc.VectorSubcoreMesh(core_axis_name="c", subcore_axis_name="s"),
    out_type=jax.ShapeDtypeStruct((2, N_NODE, EMB), jnp.float32),
    scratch_types=(
        [pltpu.VMEM_SHARED((N_NODE, EMB), jnp.float32)]
        + [pltpu.VMEM((CHUNK, EMB), jnp.float32)] * NBUF
        + [pltpu.VMEM((CHUNK,), jnp.int32)] * NBUF
        + [pltpu.VMEM((CHUNK,), jnp.int32)] * NBUF
        + [pltpu.VMEM((CHUNK,), jnp.float32)] * NBUF
        + [pltpu.SemaphoreType.DMA] * (3 * NBUF)
    ),
)


def _prep_edges(a):
    return jnp.pad(a, (0, NW * EPW - E)).reshape(NW, CPW, CHUNK)


_BLK = 2000


def _enc_body(x_ref, w1_ref, b1_ref, w2_ref, b2_ref, o_ref):
    t = _leaky(jnp.dot(x_ref[...], w1_ref[...],
                       preferred_element_type=jnp.float32) + b1_ref[...])
    o_ref[...] = jnp.dot(t, w2_ref[...],
                         preferred_element_type=jnp.float32) + b2_ref[...]


def _encoder(x, w1, b1, w2, b2):
    return pl.pallas_call(
        _enc_body,
        grid=(N_NODE // _BLK,),
        in_specs=[
            pl.BlockSpec((_BLK, EMB), lambda i: (i, 0)),
            pl.BlockSpec((EMB, EMB), lambda i: (0, 0)),
            pl.BlockSpec((1, EMB), lambda i: (0, 0)),
            pl.BlockSpec((EMB, EMB), lambda i: (0, 0)),
            pl.BlockSpec((1, EMB), lambda i: (0, 0)),
        ],
        out_specs=pl.BlockSpec((_BLK, EMB), lambda i: (i, 0)),
        out_shape=jax.ShapeDtypeStruct((N_NODE, EMB), jnp.float32),
    )(x, w1, b1.reshape(1, EMB), w2, b2.reshape(1, EMB))


def _upd_body(h_ref, p0_ref, p1_ref, w_ref, b_ref, g_ref, bb_ref,
              raw_ref, act_ref):
    pre = h_ref[...] + p0_ref[...] + p1_ref[...]
    raw = jnp.dot(pre, w_ref[...],
                  preferred_element_type=jnp.float32) + b_ref[...]
    raw_ref[...] = raw
    mu = jnp.mean(raw, axis=-1, keepdims=True)
    var = jnp.mean((raw - mu) ** 2, axis=-1, keepdims=True)
    act_ref[...] = _leaky((raw - mu) * lax.rsqrt(var + 1e-5) * g_ref[...]
                          + bb_ref[...])


def _update(h, p0, p1, w, b, g, bb):
    return pl.pallas_call(
        _upd_body,
        grid=(N_NODE // _BLK,),
        in_specs=[
            pl.BlockSpec((_BLK, EMB), lambda i: (i, 0)),
            pl.BlockSpec((_BLK, EMB), lambda i: (i, 0)),
            pl.BlockSpec((_BLK, EMB), lambda i: (i, 0)),
            pl.BlockSpec((EMB, EMB), lambda i: (0, 0)),
            pl.BlockSpec((1, EMB), lambda i: (0, 0)),
            pl.BlockSpec((1, EMB), lambda i: (0, 0)),
            pl.BlockSpec((1, EMB), lambda i: (0, 0)),
        ],
        out_specs=[
            pl.BlockSpec((_BLK, EMB), lambda i: (i, 0)),
            pl.BlockSpec((_BLK, EMB), lambda i: (i, 0)),
        ],
        out_shape=[
            jax.ShapeDtypeStruct((N_NODE, EMB), jnp.float32),
            jax.ShapeDtypeStruct((N_NODE, EMB), jnp.float32),
        ],
    )(h, p0, p1, w, b.reshape(1, EMB), g.reshape(1, EMB), bb.reshape(1, EMB))


def _head_body(h_ref, w1_ref, b1_ref, w2_ref, b2_ref, o_ref):
    t = _leaky(jnp.dot(h_ref[...], w1_ref[...],
                       preferred_element_type=jnp.float32) + b1_ref[...])
    o_ref[...] = jnp.dot(t, w2_ref[...],
                         preferred_element_type=jnp.float32) + b2_ref[...]


def _head(h, w1, b1, w2, b2):
    w2p = jnp.pad(w2, ((0, 0), (0, EMB - OUT)))
    b2p = jnp.pad(b2, (0, EMB - OUT))
    return pl.pallas_call(
        _head_body,
        grid=(N_NODE // _BLK,),
        in_specs=[
            pl.BlockSpec((_BLK, EMB), lambda i: (i, 0)),
            pl.BlockSpec((EMB, 256), lambda i: (0, 0)),
            pl.BlockSpec((1, 256), lambda i: (0, 0)),
            pl.BlockSpec((256, EMB), lambda i: (0, 0)),
            pl.BlockSpec((1, EMB), lambda i: (0, 0)),
        ],
        out_specs=pl.BlockSpec((_BLK, EMB), lambda i: (i, 0)),
        out_shape=jax.ShapeDtypeStruct((N_NODE, EMB), jnp.float32),
    )(h, w1, b1.reshape(1, 256), w2p, b2p.reshape(1, EMB))


def kernel(node_x, net_x, edge_index_n2n, edge_weight_n2n,
           edge_index_net2node, edge_weight_net2node,
           Wn1, bn1, Wn2, bn2, Wt1, bt1, Wt2, bt2,
           conv_Wnet, conv_bnet, conv_Wnode, conv_bnode, ln_g, ln_b,
           fc1n_W, fc1n_b, fc2n_W, fc2n_b, fc1t_W, fc1t_b, fc2t_W, fc2t_b):
    h_inst = _encoder(node_x, Wn1, bn1, Wn2, bn2)
    h_net = _encoder(net_x, Wt1, bt1, Wt2, bt2)

    src_n = _prep_edges(edge_index_n2n[0])
    dst_net = _prep_edges(edge_index_n2n[1])
    w_n2n = _prep_edges(edge_weight_n2n)
    src_net = _prep_edges(edge_index_net2node[0])
    dst_n = _prep_edges(edge_index_net2node[1])
    w_net2n = _prep_edges(edge_weight_net2node)

    zeros = jnp.zeros((N_NODE, EMB), jnp.float32)

    for l in range(L):
        agg_net = _spmm(h_inst, src_n, dst_net, w_n2n, zeros)
        h_net_raw, h_net_act = _update(h_net, agg_net[0], agg_net[1],
                                       conv_Wnet[l], conv_bnet[l],
                                       ln_g[l], ln_b[l])
        agg_node = _spmm(h_net_raw, src_net, dst_n, w_net2n, zeros)
        h_inst_raw, h_inst_act = _update(h_inst, agg_node[0], agg_node[1],
                                         conv_Wnode[l], conv_bnode[l],
                                         ln_g[l], ln_b[l])
        h_inst, h_net = h_inst_act, h_net_act

    out_n = _head(h_inst, fc1n_W, fc1n_b, fc2n_W, fc2n_b)[:, :OUT]
    out_t = _head(h_net, fc1t_W, fc1t_b, fc2t_W, fc2t_b)[:, :OUT]
    return (out_n, out_t)


# speedup vs baseline: 1.4319x; 1.4319x over previous
import functools

import jax
import jax.numpy as jnp
from jax import lax
from jax.experimental import pallas as pl
from jax.experimental.pallas import tpu as pltpu
from jax.experimental.pallas import tpu_sc as plsc

L = 3
EMB = 128
N_NODE = 10000
N_NET = 10000
E = 320000
OUT = 8

NW = 32
CHUNK = 128
CPW = 79
EPW = CPW * CHUNK
STRIPE = 624


def _leaky(x):
    return jnp.where(x >= 0, x, 0.01 * x)


def _spmm_body(h_hbm, src_hbm, dst_hbm, w_hbm, zeros_hbm, out_hbm,
               acc, idx_v, dst_v, w_v, rows_v, sem):
    cid = lax.axis_index("c")
    sid = lax.axis_index("s")
    wid = cid * 16 + sid

    @pl.when(sid < 15)
    def _():
        pltpu.sync_copy(zeros_hbm.at[pl.ds(sid * STRIPE, STRIPE)],
                        acc.at[pl.ds(sid * STRIPE, STRIPE)])

    @pl.when(sid == 15)
    def _():
        pltpu.sync_copy(zeros_hbm.at[pl.ds(15 * STRIPE, N_NODE - 15 * STRIPE)],
                        acc.at[pl.ds(15 * STRIPE, N_NODE - 15 * STRIPE)])

    plsc.subcore_barrier()

    def chunk_body(i, carry):
        pltpu.sync_copy(src_hbm.at[wid, i], idx_v)
        pltpu.sync_copy(dst_hbm.at[wid, i], dst_v)
        pltpu.sync_copy(w_hbm.at[wid, i], w_v)
        pltpu.async_copy(h_hbm.at[idx_v], rows_v, sem).wait()

        def grp_body(g, c2):
            wg = w_v[pl.ds(g * 16, 16)]
            for e16 in range(16):
                e = g * 16 + e16
                wb = wg[e16]
                for cb in range(EMB // 16):
                    sl = pl.ds(cb * 16, 16)
                    rows_v[e, sl] = rows_v[e, sl] * wb
            return c2
        lax.fori_loop(0, CHUNK // 16, grp_body, 0)

        pltpu.sync_copy(rows_v, acc.at[dst_v], add=True)
        return carry

    lax.fori_loop(0, CPW, chunk_body, 0)
    plsc.subcore_barrier()

    @pl.when(sid < 15)
    def _():
        pltpu.sync_copy(acc.at[pl.ds(sid * STRIPE, STRIPE)],
                        out_hbm.at[cid, pl.ds(sid * STRIPE, STRIPE)])

    @pl.when(sid == 15)
    def _():
        pltpu.sync_copy(acc.at[pl.ds(15 * STRIPE, N_NODE - 15 * STRIPE)],
                        out_hbm.at[cid, pl.ds(15 * STRIPE, N_NODE - 15 * STRIPE)])


_spmm = pl.kernel(
    _spmm_body,
    mesh=plsc.VectorSubcoreMesh(core_axis_name="c", subcore_axis_name="s"),
    out_type=jax.ShapeDtypeStruct((2, N_NODE, EMB), jnp.float32),
    scratch_types=[
        pltpu.VMEM_SHARED((N_NODE, EMB), jnp.float32),
        pltpu.VMEM((CHUNK,), jnp.int32),
        pltpu.VMEM((CHUNK,), jnp.int32),
        pltpu.VMEM((CHUNK,), jnp.float32),
        pltpu.VMEM((CHUNK, EMB), jnp.float32),
        pltpu.SemaphoreType.DMA,
    ],
)


def _prep_edges(a):
    return jnp.pad(a, (0, NW * EPW - E)).reshape(NW, CPW, CHUNK)


_BLK = 2000


def _enc_body(x_ref, w1_ref, b1_ref, w2_ref, b2_ref, o_ref):
    t = _leaky(jnp.dot(x_ref[...], w1_ref[...],
                       preferred_element_type=jnp.float32) + b1_ref[...])
    o_ref[...] = jnp.dot(t, w2_ref[...],
                         preferred_element_type=jnp.float32) + b2_ref[...]


def _encoder(x, w1, b1, w2, b2):
    return pl.pallas_call(
        _enc_body,
        grid=(N_NODE // _BLK,),
        in_specs=[
            pl.BlockSpec((_BLK, EMB), lambda i: (i, 0)),
            pl.BlockSpec((EMB, EMB), lambda i: (0, 0)),
            pl.BlockSpec((1, EMB), lambda i: (0, 0)),
            pl.BlockSpec((EMB, EMB), lambda i: (0, 0)),
            pl.BlockSpec((1, EMB), lambda i: (0, 0)),
        ],
        out_specs=pl.BlockSpec((_BLK, EMB), lambda i: (i, 0)),
        out_shape=jax.ShapeDtypeStruct((N_NODE, EMB), jnp.float32),
    )(x, w1, b1.reshape(1, EMB), w2, b2.reshape(1, EMB))


def _upd_body(h_ref, p0_ref, p1_ref, w_ref, b_ref, g_ref, bb_ref,
              raw_ref, act_ref):
    pre = h_ref[...] + p0_ref[...] + p1_ref[...]
    raw = jnp.dot(pre, w_ref[...],
                  preferred_element_type=jnp.float32) + b_ref[...]
    raw_ref[...] = raw
    mu = jnp.mean(raw, axis=-1, keepdims=True)
    var = jnp.mean((raw - mu) ** 2, axis=-1, keepdims=True)
    act_ref[...] = _leaky((raw - mu) * lax.rsqrt(var + 1e-5) * g_ref[...]
                          + bb_ref[...])


def _update(h, p0, p1, w, b, g, bb):
    return pl.pallas_call(
        _upd_body,
        grid=(N_NODE // _BLK,),
        in_specs=[
            pl.BlockSpec((_BLK, EMB), lambda i: (i, 0)),
            pl.BlockSpec((_BLK, EMB), lambda i: (i, 0)),
            pl.BlockSpec((_BLK, EMB), lambda i: (i, 0)),
            pl.BlockSpec((EMB, EMB), lambda i: (0, 0)),
            pl.BlockSpec((1, EMB), lambda i: (0, 0)),
            pl.BlockSpec((1, EMB), lambda i: (0, 0)),
            pl.BlockSpec((1, EMB), lambda i: (0, 0)),
        ],
        out_specs=[
            pl.BlockSpec((_BLK, EMB), lambda i: (i, 0)),
            pl.BlockSpec((_BLK, EMB), lambda i: (i, 0)),
        ],
        out_shape=[
            jax.ShapeDtypeStruct((N_NODE, EMB), jnp.float32),
            jax.ShapeDtypeStruct((N_NODE, EMB), jnp.float32),
        ],
    )(h, p0, p1, w, b.reshape(1, EMB), g.reshape(1, EMB), bb.reshape(1, EMB))


def _head_body(h_ref, w1_ref, b1_ref, w2_ref, b2_ref, o_ref):
    t = _leaky(jnp.dot(h_ref[...], w1_ref[...],
                       preferred_element_type=jnp.float32) + b1_ref[...])
    o_ref[...] = jnp.dot(t, w2_ref[...],
                         preferred_element_type=jnp.float32) + b2_ref[...]


def _head(h, w1, b1, w2, b2):
    w2p = jnp.pad(w2, ((0, 0), (0, EMB - OUT)))
    b2p = jnp.pad(b2, (0, EMB - OUT))
    return pl.pallas_call(
        _head_body,
        grid=(N_NODE // _BLK,),
        in_specs=[
            pl.BlockSpec((_BLK, EMB), lambda i: (i, 0)),
            pl.BlockSpec((EMB, 256), lambda i: (0, 0)),
            pl.BlockSpec((1, 256), lambda i: (0, 0)),
            pl.BlockSpec((256, EMB), lambda i: (0, 0)),
            pl.BlockSpec((1, EMB), lambda i: (0, 0)),
        ],
        out_specs=pl.BlockSpec((_BLK, EMB), lambda i: (i, 0)),
        out_shape=jax.ShapeDtypeStruct((N_NODE, EMB), jnp.float32),
    )(h, w1, b1.reshape(1, 256), w2p, b2p.reshape(1, EMB))


def kernel(node_x, net_x, edge_index_n2n, edge_weight_n2n,
           edge_index_net2node, edge_weight_net2node,
           Wn1, bn1, Wn2, bn2, Wt1, bt1, Wt2, bt2,
           conv_Wnet, conv_bnet, conv_Wnode, conv_bnode, ln_g, ln_b,
           fc1n_W, fc1n_b, fc2n_W, fc2n_b, fc1t_W, fc1t_b, fc2t_W, fc2t_b):
    h_inst = _encoder(node_x, Wn1, bn1, Wn2, bn2)
    h_net = _encoder(net_x, Wt1, bt1, Wt2, bt2)

    src_n = _prep_edges(edge_index_n2n[0])
    dst_net = _prep_edges(edge_index_n2n[1])
    w_n2n = _prep_edges(edge_weight_n2n)
    src_net = _prep_edges(edge_index_net2node[0])
    dst_n = _prep_edges(edge_index_net2node[1])
    w_net2n = _prep_edges(edge_weight_net2node)

    zeros = jnp.zeros((N_NODE, EMB), jnp.float32)

    for l in range(L):
        agg_net = _spmm(h_inst, src_n, dst_net, w_n2n, zeros)
        h_net_raw, h_net_act = _update(h_net, agg_net[0], agg_net[1],
                                       conv_Wnet[l], conv_bnet[l],
                                       ln_g[l], ln_b[l])
        agg_node = _spmm(h_net_raw, src_net, dst_n, w_net2n, zeros)
        h_inst_raw, h_inst_act = _update(h_inst, agg_node[0], agg_node[1],
                                         conv_Wnode[l], conv_bnode[l],
                                         ln_g[l], ln_b[l])
        h_inst, h_net = h_inst_act, h_net_act

    out_n = _head(h_inst, fc1n_W, fc1n_b, fc2n_W, fc2n_b)[:, :OUT]
    out_t = _head(h_net, fc1t_W, fc1t_b, fc2t_W, fc2t_b)[:, :OUT]
    return (out_n, out_t)
